# Initial kernel scaffold; baseline (speedup 1.0000x reference)
import jax
import jax.numpy as jnp
from jax.experimental import pallas as pl


def kernel(x1, x2, x3, table, t1_bias1, t1_W, t1_b, t2_bias1, t2_W, t2_b):
    raise NotImplementedError("write your pallas kernel here")


# speedup vs baseline: 6.1664x; 6.1664x over previous
import jax
import jax.numpy as jnp
from jax import lax
from jax.experimental import pallas as pl
from jax.experimental.pallas import tpu as pltpu
from jax.experimental.pallas import tpu_sc as plsc

D = 32
B = 16384
H = 200
NC, NS, L = 2, 16, 16
NW = NC * NS
BPW = B // NW


def _lane_bcast(vec, j):
    idx = jnp.full((L, 1), j, jnp.int32)
    return lax.gather(
        vec, idx,
        lax.GatherDimensionNumbers(offset_dims=(), collapsed_slice_dims=(0,),
                                   start_index_map=(0,)),
        (1,), mode=lax.GatherScatterMode.PROMISE_IN_BOUNDS)


def _pool_body(x1_hbm, x2_hbm, x3_hbm, table_hbm, p1_hbm, p2_hbm,
               idx1_v, idx2_v, w_v, rows1_v, rows2_v, out1_v, out2_v, sem):
    cid = lax.axis_index("c")
    sid = lax.axis_index("s")
    wid = sid * NC + cid
    base = wid * BPW

    def row_body(r, carry):
        b = base + r
        pltpu.sync_copy(x1_hbm.at[b, pl.ds(0, 128)], idx1_v.at[0])
        pltpu.sync_copy(x1_hbm.at[b, pl.ds(128, 72)], idx1_v.at[1, pl.ds(0, 72)])
        pltpu.sync_copy(x2_hbm.at[b, pl.ds(0, 128)], idx2_v.at[0])
        pltpu.sync_copy(x2_hbm.at[b, pl.ds(128, 72)], idx2_v.at[1, pl.ds(0, 72)])
        pltpu.sync_copy(x3_hbm.at[b], w_v)
        g1 = pltpu.async_copy(table_hbm.at[idx1_v.at[0]], rows1_v.at[pl.ds(0, 128)], sem)
        g2 = pltpu.async_copy(table_hbm.at[idx1_v.at[1, pl.ds(0, 72)]], rows1_v.at[pl.ds(128, 72)], sem)
        g3 = pltpu.async_copy(table_hbm.at[idx2_v.at[0]], rows2_v.at[pl.ds(0, 128)], sem)
        g4 = pltpu.async_copy(table_hbm.at[idx2_v.at[1, pl.ds(0, 72)]], rows2_v.at[pl.ds(128, 72)], sem)
        g1.wait(); g2.wait(); g3.wait(); g4.wait()

        a0 = jnp.zeros((L,), jnp.float32)
        a1 = jnp.zeros((L,), jnp.float32)
        a2 = jnp.zeros((L,), jnp.float32)
        a3 = jnp.zeros((L,), jnp.float32)
        for c in range(13):
            eb = 184 if c == 12 else c * 16
            w = w_v[pl.ds(eb, L)]
            for j in (range(8, 16) if c == 12 else range(16)):
                i = eb + j
                wj = _lane_bcast(w, j)
                a0 = a0 + wj * rows1_v[i, pl.ds(0, L)]
                a1 = a1 + wj * rows1_v[i, pl.ds(L, L)]
                a2 = a2 + rows2_v[i, pl.ds(0, L)]
                a3 = a3 + rows2_v[i, pl.ds(L, L)]
        out1_v[r, pl.ds(0, L)] = a0
        out1_v[r, pl.ds(L, L)] = a1
        out2_v[r, pl.ds(0, L)] = a2
        out2_v[r, pl.ds(L, L)] = a3
        return carry

    lax.fori_loop(0, BPW, row_body, 0)
    pltpu.sync_copy(out1_v, p1_hbm.at[pl.ds(base, BPW)])
    pltpu.sync_copy(out2_v, p2_hbm.at[pl.ds(base, BPW)])


_pool = pl.kernel(
    _pool_body,
    out_type=(jax.ShapeDtypeStruct((B, D), jnp.float32),
              jax.ShapeDtypeStruct((B, D), jnp.float32)),
    mesh=plsc.VectorSubcoreMesh(core_axis_name="c", subcore_axis_name="s",
                                num_cores=NC, num_subcores=NS),
    scratch_types=[
        pltpu.VMEM((2, 128), jnp.int32),
        pltpu.VMEM((2, 128), jnp.int32),
        pltpu.VMEM((H,), jnp.float32),
        pltpu.VMEM((H, D), jnp.float32),
        pltpu.VMEM((H, D), jnp.float32),
        pltpu.VMEM((BPW, D), jnp.float32),
        pltpu.VMEM((BPW, D), jnp.float32),
        pltpu.SemaphoreType.DMA,
    ],
    compiler_params=pltpu.CompilerParams(use_tc_tiling_on_sc=False),
)


def _mlp_body(p1_ref, p2_ref, b1_ref, w1_ref, c1_ref, b2_ref, w2_ref, c2_ref,
              o_ref):
    v1 = jnp.tanh(p1_ref[...] + b1_ref[...])
    v1 = jnp.tanh(
        lax.dot_general(v1, w1_ref[...], (((1,), (1,)), ((), ())),
                        preferred_element_type=jnp.float32) + c1_ref[...])
    v2 = jnp.tanh(p2_ref[...] + b2_ref[...])
    v2 = jnp.tanh(
        lax.dot_general(v2, w2_ref[...], (((1,), (1,)), ((), ())),
                        preferred_element_type=jnp.float32) + c2_ref[...])
    o_ref[...] = jax.nn.sigmoid(jnp.sum(v1 * v2, axis=1))


_mlp = pl.pallas_call(
    _mlp_body,
    out_shape=jax.ShapeDtypeStruct((B,), jnp.float32),
)


@jax.jit
def kernel(x1, x2, x3, table, t1_bias1, t1_W, t1_b, t2_bias1, t2_W, t2_b):
    p1, p2 = _pool(x1, x2, x3, table)
    return _mlp(p1, p2, t1_bias1.reshape(1, D), t1_W, t1_b.reshape(1, D),
                t2_bias1.reshape(1, D), t2_W, t2_b.reshape(1, D))


# speedup vs baseline: 15.8072x; 2.5634x over previous
import jax
import jax.numpy as jnp
from jax import lax
from jax.experimental import pallas as pl
from jax.experimental.pallas import tpu as pltpu
from jax.experimental.pallas import tpu_sc as plsc

D = 32
B = 16384
H = 200
NC, NS, L = 2, 16, 16
NW = NC * NS
BPW = B // NW
CH = 16
NCH = BPW // CH


def _lane_bcast(vec, jvec):
    return lax.gather(
        vec, jvec,
        lax.GatherDimensionNumbers(offset_dims=(), collapsed_slice_dims=(0,),
                                   start_index_map=(0,)),
        (1,), mode=lax.GatherScatterMode.PROMISE_IN_BOUNDS)


def _pool_body(x1_hbm, x2_hbm, x3_hbm, table_hbm, p1_hbm, p2_hbm,
               x1s, x2s, x3s, rows1_v, rows2_v, out1_v, out2_v,
               sem_stage, sem_g):
    cid = lax.axis_index("c")
    sid = lax.axis_index("s")
    wid = sid * NC + cid
    base = wid * BPW

    jv = [jnp.full((L, 1), j, jnp.int32) for j in range(L)]

    def stage_issue(c, cb):
        b0 = base + c * CH
        pltpu.async_copy(x1_hbm.at[pl.ds(b0, CH)], x1s.at[cb], sem_stage.at[cb])
        pltpu.async_copy(x2_hbm.at[pl.ds(b0, CH)], x2s.at[cb], sem_stage.at[cb])
        pltpu.async_copy(x3_hbm.at[pl.ds(b0, CH)], x3s.at[cb], sem_stage.at[cb])

    def stage_wait(cb):
        pltpu.make_async_copy(x1_hbm.at[pl.ds(0, CH)], x1s.at[cb], sem_stage.at[cb]).wait()
        pltpu.make_async_copy(x2_hbm.at[pl.ds(0, CH)], x2s.at[cb], sem_stage.at[cb]).wait()
        pltpu.make_async_copy(x3_hbm.at[pl.ds(0, CH)], x3s.at[cb], sem_stage.at[cb]).wait()

    def gather_issue(r, p):
        c = r // CH
        cb = c % 2
        rr = r % CH
        pltpu.async_copy(table_hbm.at[x1s.at[cb, rr, pl.ds(0, 128)]],
                         rows1_v.at[p, pl.ds(0, 128)], sem_g.at[p])
        pltpu.async_copy(table_hbm.at[x1s.at[cb, rr, pl.ds(128, 72)]],
                         rows1_v.at[p, pl.ds(128, 72)], sem_g.at[p])
        pltpu.async_copy(table_hbm.at[x2s.at[cb, rr, pl.ds(0, 128)]],
                         rows2_v.at[p, pl.ds(0, 128)], sem_g.at[p])
        pltpu.async_copy(table_hbm.at[x2s.at[cb, rr, pl.ds(128, 72)]],
                         rows2_v.at[p, pl.ds(128, 72)], sem_g.at[p])

    def gather_wait(p):
        pltpu.make_async_copy(table_hbm.at[pl.ds(0, 128)],
                              rows1_v.at[p, pl.ds(0, 128)], sem_g.at[p]).wait()
        pltpu.make_async_copy(table_hbm.at[pl.ds(0, 72)],
                              rows1_v.at[p, pl.ds(128, 72)], sem_g.at[p]).wait()
        pltpu.make_async_copy(table_hbm.at[pl.ds(0, 128)],
                              rows2_v.at[p, pl.ds(0, 128)], sem_g.at[p]).wait()
        pltpu.make_async_copy(table_hbm.at[pl.ds(0, 72)],
                              rows2_v.at[p, pl.ds(128, 72)], sem_g.at[p]).wait()

    stage_issue(0, 0)
    stage_issue(1, 1)
    stage_wait(0)
    gather_issue(0, 0)

    def row_body(r, carry):
        p = r % 2
        c = r // CH
        cb = c % 2
        rr = r % CH

        gather_wait(p)

        last_in_chunk = rr == CH - 1
        not_last_row = r < BPW - 1

        @pl.when(jnp.logical_and(last_in_chunk, not_last_row))
        def _():
            stage_wait((c + 1) % 2)
            gather_issue(r + 1, 1 - p)

        @pl.when(jnp.logical_and(jnp.logical_not(last_in_chunk), not_last_row))
        def _():
            gather_issue(r + 1, 1 - p)

        a0 = jnp.zeros((L,), jnp.float32)
        a1 = jnp.zeros((L,), jnp.float32)
        a2 = jnp.zeros((L,), jnp.float32)
        a3 = jnp.zeros((L,), jnp.float32)
        for ci in range(13):
            eb = 184 if ci == 12 else ci * 16
            w = x3s[cb, rr, pl.ds(eb, L)]
            for j in (range(8, 16) if ci == 12 else range(16)):
                i = eb + j
                wj = _lane_bcast(w, jv[j])
                a0 = a0 + wj * rows1_v[p, i, pl.ds(0, L)]
                a1 = a1 + wj * rows1_v[p, i, pl.ds(L, L)]
                a2 = a2 + rows2_v[p, i, pl.ds(0, L)]
                a3 = a3 + rows2_v[p, i, pl.ds(L, L)]
        out1_v[r, pl.ds(0, L)] = a0
        out1_v[r, pl.ds(L, L)] = a1
        out2_v[r, pl.ds(0, L)] = a2
        out2_v[r, pl.ds(L, L)] = a3

        @pl.when(jnp.logical_and(last_in_chunk, c + 2 < NCH))
        def _():
            stage_issue(c + 2, cb)

        return carry

    lax.fori_loop(0, BPW, row_body, 0)
    pltpu.sync_copy(out1_v, p1_hbm.at[pl.ds(base, BPW)])
    pltpu.sync_copy(out2_v, p2_hbm.at[pl.ds(base, BPW)])


_pool = pl.kernel(
    _pool_body,
    out_type=(jax.ShapeDtypeStruct((B, D), jnp.float32),
              jax.ShapeDtypeStruct((B, D), jnp.float32)),
    mesh=plsc.VectorSubcoreMesh(core_axis_name="c", subcore_axis_name="s",
                                num_cores=NC, num_subcores=NS),
    scratch_types=[
        pltpu.VMEM((2, CH, H), jnp.int32),
        pltpu.VMEM((2, CH, H), jnp.int32),
        pltpu.VMEM((2, CH, H), jnp.float32),
        pltpu.VMEM((2, H, D), jnp.float32),
        pltpu.VMEM((2, H, D), jnp.float32),
        pltpu.VMEM((BPW, D), jnp.float32),
        pltpu.VMEM((BPW, D), jnp.float32),
        pltpu.SemaphoreType.DMA((2,)),
        pltpu.SemaphoreType.DMA((2,)),
    ],
    compiler_params=pltpu.CompilerParams(use_tc_tiling_on_sc=False),
)


def _mlp_body(p1_ref, p2_ref, b1_ref, w1_ref, c1_ref, b2_ref, w2_ref, c2_ref,
              o_ref):
    v1 = jnp.tanh(p1_ref[...] + b1_ref[...])
    v1 = jnp.tanh(
        lax.dot_general(v1, w1_ref[...], (((1,), (1,)), ((), ())),
                        preferred_element_type=jnp.float32) + c1_ref[...])
    v2 = jnp.tanh(p2_ref[...] + b2_ref[...])
    v2 = jnp.tanh(
        lax.dot_general(v2, w2_ref[...], (((1,), (1,)), ((), ())),
                        preferred_element_type=jnp.float32) + c2_ref[...])
    o_ref[...] = jax.nn.sigmoid(jnp.sum(v1 * v2, axis=1))


_mlp = pl.pallas_call(
    _mlp_body,
    out_shape=jax.ShapeDtypeStruct((B,), jnp.float32),
)


@jax.jit
def kernel(x1, x2, x3, table, t1_bias1, t1_W, t1_b, t2_bias1, t2_W, t2_b):
    p1, p2 = _pool(x1, x2, x3, table)
    return _mlp(p1, p2, t1_bias1.reshape(1, D), t1_W, t1_b.reshape(1, D),
                t2_bias1.reshape(1, D), t2_W, t2_b.reshape(1, D))
